# Initial kernel scaffold; baseline (speedup 1.0000x reference)
import jax
import jax.numpy as jnp
from jax.experimental import pallas as pl


def kernel(unknown, known, unknow_feats, known_feats, W1, g1, b1, W2, g2, b2):
    raise NotImplementedError("write your pallas kernel here")


# speedup vs baseline: 18.7135x; 18.7135x over previous
import functools

import jax
import jax.numpy as jnp
from jax.experimental import pallas as pl


_TN = 512


def _pass1_kernel(unk_ref, known_ref, kf_ref, uf_ref, w1_ref,
                  x1_ref, acc_ref, *, m):
    b = pl.program_id(0)
    t = pl.program_id(1)

    tn = unk_ref.shape[2]
    cross = jnp.zeros((m, tn), dtype=jnp.float32)
    u2 = jnp.zeros((1, tn), dtype=jnp.float32)
    k2 = jnp.zeros((m, 1), dtype=jnp.float32)
    for d in range(3):
        kd = known_ref[0, :, d:d + 1]
        ud = unk_ref[0, d:d + 1, :]
        kb = kd.astype(jnp.bfloat16).astype(jnp.float32)
        ub = ud.astype(jnp.bfloat16).astype(jnp.float32)
        cross = cross + kb * ub
        u2 = u2 + ud * ud
        k2 = k2 + kd * kd
    d2 = jnp.maximum((u2 + k2) - 2.0 * cross, 0.0)

    miota = jax.lax.broadcasted_iota(jnp.int32, (m, tn), 0)
    big = jnp.float32(jnp.inf)
    dcur = d2
    dists = []
    idxs = []
    for _ in range(3):
        mk = jnp.min(dcur, axis=0, keepdims=True)
        ik = jnp.min(jnp.where(dcur == mk, miota, m),
                     axis=0, keepdims=True)
        dists.append(mk)
        idxs.append(ik)
        dcur = jnp.where(miota == ik, big, dcur)

    r0 = 1.0 / (dists[0] + 1e-8)
    r1 = 1.0 / (dists[1] + 1e-8)
    r2 = 1.0 / (dists[2] + 1e-8)
    rnorm = 1.0 / (r0 + r1 + r2)
    w0 = r0 * rnorm
    w1 = r1 * rnorm
    w2 = r2 * rnorm

    zero = jnp.zeros((m, tn), dtype=jnp.float32)
    st = jnp.where(miota == idxs[0], w0, zero)
    st = jnp.where(miota == idxs[1], w1, st)
    st = jnp.where(miota == idxs[2], w2, st)

    interp = jnp.dot(kf_ref[0], st, preferred_element_type=jnp.float32)
    f = jnp.concatenate([interp, uf_ref[0]], axis=0)
    x1 = jnp.dot(w1_ref[...], f, preferred_element_type=jnp.float32)
    x1_ref[0] = x1

    @pl.when(jnp.logical_and(b == 0, t == 0))
    def _init():
        acc_ref[...] = jnp.zeros_like(acc_ref)

    rs = jnp.sum(x1, axis=1, keepdims=True)
    rss = jnp.sum(x1 * x1, axis=1, keepdims=True)
    acc_ref[...] += jnp.concatenate([rs, rss], axis=1)


def _pass2_kernel(x1_ref, sc_ref, bi_ref, w2_ref, x2_ref, acc_ref):
    b = pl.program_id(0)
    t = pl.program_id(1)
    y = jnp.maximum(x1_ref[0] * sc_ref[...] + bi_ref[...], 0.0)
    x2 = jnp.dot(w2_ref[...], y, preferred_element_type=jnp.float32)
    x2_ref[0] = x2

    @pl.when(jnp.logical_and(b == 0, t == 0))
    def _init():
        acc_ref[...] = jnp.zeros_like(acc_ref)

    rs = jnp.sum(x2, axis=1, keepdims=True)
    rss = jnp.sum(x2 * x2, axis=1, keepdims=True)
    acc_ref[...] += jnp.concatenate([rs, rss], axis=1)


def _pass3_kernel(x2_ref, sc_ref, bi_ref, out_ref):
    out_ref[0] = jnp.maximum(x2_ref[0] * sc_ref[...] + bi_ref[...], 0.0)


def _bn_coeffs(acc, ntot, g, b, eps=1e-5):
    mean = acc[:, 0] / ntot
    var = acc[:, 1] / ntot - mean * mean
    sc = g / jnp.sqrt(var + eps)
    bi = b - mean * sc
    return sc[:, None], bi[:, None]


@functools.partial(jax.jit, static_argnames=())
def kernel(unknown, known, unknow_feats, known_feats, W1, g1, b1, W2, g2, b2):
    B, n, _ = unknown.shape
    m = known.shape[1]
    c2 = known_feats.shape[1]
    c1 = unknow_feats.shape[1]
    co1 = W1.shape[0]
    co2 = W2.shape[0]
    tn = _TN
    nt = n // tn
    ntot = jnp.float32(B * n)

    unk_t = jnp.transpose(unknown, (0, 2, 1))

    grid = (B, nt)
    x1, acc1 = pl.pallas_call(
        functools.partial(_pass1_kernel, m=m),
        grid=grid,
        in_specs=[
            pl.BlockSpec((1, 3, tn), lambda b, t: (b, 0, t)),
            pl.BlockSpec((1, m, 3), lambda b, t: (b, 0, 0)),
            pl.BlockSpec((1, c2, m), lambda b, t: (b, 0, 0)),
            pl.BlockSpec((1, c1, tn), lambda b, t: (b, 0, t)),
            pl.BlockSpec((co1, c1 + c2), lambda b, t: (0, 0)),
        ],
        out_specs=[
            pl.BlockSpec((1, co1, tn), lambda b, t: (b, 0, t)),
            pl.BlockSpec((co1, 2), lambda b, t: (0, 0)),
        ],
        out_shape=[
            jax.ShapeDtypeStruct((B, co1, n), jnp.float32),
            jax.ShapeDtypeStruct((co1, 2), jnp.float32),
        ],
    )(unk_t, known, known_feats, unknow_feats, W1)

    sc1, bi1 = _bn_coeffs(acc1, ntot, g1, b1)

    x2, acc2 = pl.pallas_call(
        _pass2_kernel,
        grid=grid,
        in_specs=[
            pl.BlockSpec((1, co1, tn), lambda b, t: (b, 0, t)),
            pl.BlockSpec((co1, 1), lambda b, t: (0, 0)),
            pl.BlockSpec((co1, 1), lambda b, t: (0, 0)),
            pl.BlockSpec((co2, co1), lambda b, t: (0, 0)),
        ],
        out_specs=[
            pl.BlockSpec((1, co2, tn), lambda b, t: (b, 0, t)),
            pl.BlockSpec((co2, 2), lambda b, t: (0, 0)),
        ],
        out_shape=[
            jax.ShapeDtypeStruct((B, co2, n), jnp.float32),
            jax.ShapeDtypeStruct((co2, 2), jnp.float32),
        ],
    )(x1, sc1, bi1, W2)

    sc2, bi2 = _bn_coeffs(acc2, ntot, g2, b2)

    out = pl.pallas_call(
        _pass3_kernel,
        grid=grid,
        in_specs=[
            pl.BlockSpec((1, co2, tn), lambda b, t: (b, 0, t)),
            pl.BlockSpec((co2, 1), lambda b, t: (0, 0)),
            pl.BlockSpec((co2, 1), lambda b, t: (0, 0)),
        ],
        out_specs=pl.BlockSpec((1, co2, tn), lambda b, t: (b, 0, t)),
        out_shape=jax.ShapeDtypeStruct((B, co2, n), jnp.float32),
    )(x2, sc2, bi2)

    return out


# speedup vs baseline: 19.3452x; 1.0338x over previous
import functools

import jax
import jax.numpy as jnp
from jax.experimental import pallas as pl


_TN = 512


def _pass1_kernel(unk_ref, known_ref, kf_ref, uf_ref, w1_ref,
                  x1_ref, acc_ref, *, m):
    b = pl.program_id(0)
    t = pl.program_id(1)

    tn = unk_ref.shape[2]
    kb = known_ref[0].astype(jnp.bfloat16)
    ub = unk_ref[0].astype(jnp.bfloat16)
    cross = jnp.dot(kb, ub, preferred_element_type=jnp.float32)
    u2 = jnp.zeros((1, tn), dtype=jnp.float32)
    k2 = jnp.zeros((m, 1), dtype=jnp.float32)
    for d in range(3):
        kd = known_ref[0, :, d:d + 1]
        ud = unk_ref[0, d:d + 1, :]
        u2 = u2 + ud * ud
        k2 = k2 + kd * kd
    d2 = jnp.maximum((u2 + k2) - 2.0 * cross, 0.0)

    miota = jax.lax.broadcasted_iota(jnp.int32, (m, tn), 0)
    big = jnp.float32(jnp.inf)
    dcur = d2
    dists = []
    idxs = []
    for _ in range(3):
        mk = jnp.min(dcur, axis=0, keepdims=True)
        ik = jnp.min(jnp.where(dcur == mk, miota, m),
                     axis=0, keepdims=True)
        dists.append(mk)
        idxs.append(ik)
        dcur = jnp.where(miota == ik, big, dcur)

    r0 = 1.0 / (dists[0] + 1e-8)
    r1 = 1.0 / (dists[1] + 1e-8)
    r2 = 1.0 / (dists[2] + 1e-8)
    rnorm = 1.0 / (r0 + r1 + r2)
    w0 = r0 * rnorm
    w1 = r1 * rnorm
    w2 = r2 * rnorm

    zero = jnp.zeros((m, tn), dtype=jnp.float32)
    st = jnp.where(miota == idxs[0], w0, zero)
    st = jnp.where(miota == idxs[1], w1, st)
    st = jnp.where(miota == idxs[2], w2, st)

    interp = jnp.dot(kf_ref[0], st, preferred_element_type=jnp.float32)
    f = jnp.concatenate([interp.astype(jnp.bfloat16), uf_ref[0]], axis=0)
    x1 = jnp.dot(w1_ref[...], f, preferred_element_type=jnp.float32)
    x1_ref[0] = x1

    @pl.when(jnp.logical_and(b == 0, t == 0))
    def _init():
        acc_ref[...] = jnp.zeros_like(acc_ref)

    rs = jnp.sum(x1, axis=1, keepdims=True)
    rss = jnp.sum(x1 * x1, axis=1, keepdims=True)
    acc_ref[...] += jnp.concatenate([rs, rss], axis=1)


def _pass2_kernel(x1_ref, sc_ref, bi_ref, w2_ref, x2_ref, acc_ref):
    b = pl.program_id(0)
    t = pl.program_id(1)
    y = jnp.maximum(x1_ref[0] * sc_ref[...] + bi_ref[...], 0.0)
    x2 = jnp.dot(w2_ref[...], y.astype(jnp.bfloat16),
                 preferred_element_type=jnp.float32)
    x2_ref[0] = x2

    @pl.when(jnp.logical_and(b == 0, t == 0))
    def _init():
        acc_ref[...] = jnp.zeros_like(acc_ref)

    rs = jnp.sum(x2, axis=1, keepdims=True)
    rss = jnp.sum(x2 * x2, axis=1, keepdims=True)
    acc_ref[...] += jnp.concatenate([rs, rss], axis=1)


def _pass3_kernel(x2_ref, sc_ref, bi_ref, out_ref):
    out_ref[0] = jnp.maximum(x2_ref[0] * sc_ref[...] + bi_ref[...], 0.0)


def _bn_coeffs(acc, ntot, g, b, eps=1e-5):
    mean = acc[:, 0] / ntot
    var = acc[:, 1] / ntot - mean * mean
    sc = g / jnp.sqrt(var + eps)
    bi = b - mean * sc
    return sc[:, None], bi[:, None]


@functools.partial(jax.jit, static_argnames=())
def kernel(unknown, known, unknow_feats, known_feats, W1, g1, b1, W2, g2, b2):
    B, n, _ = unknown.shape
    m = known.shape[1]
    c2 = known_feats.shape[1]
    c1 = unknow_feats.shape[1]
    co1 = W1.shape[0]
    co2 = W2.shape[0]
    tn = _TN
    nt = n // tn
    ntot = jnp.float32(B * n)

    unk_t = jnp.transpose(unknown, (0, 2, 1))
    uf_bf = unknow_feats.astype(jnp.bfloat16)
    w1_bf = W1.astype(jnp.bfloat16)
    w2_bf = W2.astype(jnp.bfloat16)

    grid = (B, nt)
    x1, acc1 = pl.pallas_call(
        functools.partial(_pass1_kernel, m=m),
        grid=grid,
        in_specs=[
            pl.BlockSpec((1, 3, tn), lambda b, t: (b, 0, t)),
            pl.BlockSpec((1, m, 3), lambda b, t: (b, 0, 0)),
            pl.BlockSpec((1, c2, m), lambda b, t: (b, 0, 0)),
            pl.BlockSpec((1, c1, tn), lambda b, t: (b, 0, t)),
            pl.BlockSpec((co1, c1 + c2), lambda b, t: (0, 0)),
        ],
        out_specs=[
            pl.BlockSpec((1, co1, tn), lambda b, t: (b, 0, t)),
            pl.BlockSpec((co1, 2), lambda b, t: (0, 0)),
        ],
        out_shape=[
            jax.ShapeDtypeStruct((B, co1, n), jnp.float32),
            jax.ShapeDtypeStruct((co1, 2), jnp.float32),
        ],
    )(unk_t, known, known_feats, uf_bf, w1_bf)

    sc1, bi1 = _bn_coeffs(acc1, ntot, g1, b1)

    x2, acc2 = pl.pallas_call(
        _pass2_kernel,
        grid=grid,
        in_specs=[
            pl.BlockSpec((1, co1, tn), lambda b, t: (b, 0, t)),
            pl.BlockSpec((co1, 1), lambda b, t: (0, 0)),
            pl.BlockSpec((co1, 1), lambda b, t: (0, 0)),
            pl.BlockSpec((co2, co1), lambda b, t: (0, 0)),
        ],
        out_specs=[
            pl.BlockSpec((1, co2, tn), lambda b, t: (b, 0, t)),
            pl.BlockSpec((co2, 2), lambda b, t: (0, 0)),
        ],
        out_shape=[
            jax.ShapeDtypeStruct((B, co2, n), jnp.float32),
            jax.ShapeDtypeStruct((co2, 2), jnp.float32),
        ],
    )(x1, sc1, bi1, w2_bf)

    sc2, bi2 = _bn_coeffs(acc2, ntot, g2, b2)

    out = pl.pallas_call(
        _pass3_kernel,
        grid=grid,
        in_specs=[
            pl.BlockSpec((1, co2, tn), lambda b, t: (b, 0, t)),
            pl.BlockSpec((co2, 1), lambda b, t: (0, 0)),
            pl.BlockSpec((co2, 1), lambda b, t: (0, 0)),
        ],
        out_specs=pl.BlockSpec((1, co2, tn), lambda b, t: (b, 0, t)),
        out_shape=jax.ShapeDtypeStruct((B, co2, n), jnp.float32),
    )(x2, sc2, bi2)

    return out


# speedup vs baseline: 23.6316x; 1.2216x over previous
import functools

import jax
import jax.numpy as jnp
from jax.experimental import pallas as pl


_TN = 512


def _pass1_kernel(unk_ref, known_ref, kf_ref, uf_ref, w1_ref,
                  x1_ref, acc_ref, *, m):
    b = pl.program_id(0)
    t = pl.program_id(1)

    tn = unk_ref.shape[2]
    kb = known_ref[0].astype(jnp.bfloat16)
    ub = unk_ref[0].astype(jnp.bfloat16)
    cross = jnp.dot(kb, ub, preferred_element_type=jnp.float32)
    u2 = jnp.zeros((1, tn), dtype=jnp.float32)
    k2 = jnp.zeros((m, 1), dtype=jnp.float32)
    for d in range(3):
        kd = known_ref[0, :, d:d + 1]
        ud = unk_ref[0, d:d + 1, :]
        u2 = u2 + ud * ud
        k2 = k2 + kd * kd
    d2 = jnp.maximum((u2 + k2) - 2.0 * cross, 0.0)

    big = jnp.float32(jnp.inf)
    m0 = jnp.min(d2, axis=0, keepdims=True)
    d1 = jnp.where(d2 == m0, big, d2)
    m1 = jnp.min(d1, axis=0, keepdims=True)
    db = jnp.where(d1 == m1, big, d1)
    m2 = jnp.min(db, axis=0, keepdims=True)

    r0 = 1.0 / (m0 + 1e-8)
    r1 = 1.0 / (m1 + 1e-8)
    r2 = 1.0 / (m2 + 1e-8)
    rnorm = 1.0 / (r0 + r1 + r2)

    zero = jnp.zeros((m, tn), dtype=jnp.float32)
    st = jnp.where(d2 == m0, r0, zero)
    st = jnp.where(d1 == m1, r1, st)
    st = jnp.where(db == m2, r2, st)
    st = st * rnorm

    interp = jnp.dot(kf_ref[0], st, preferred_element_type=jnp.float32)
    f = jnp.concatenate([interp.astype(jnp.bfloat16), uf_ref[0]], axis=0)
    x1 = jnp.dot(w1_ref[...], f, preferred_element_type=jnp.float32)
    x1_ref[0] = x1.astype(jnp.bfloat16)

    @pl.when(jnp.logical_and(b == 0, t == 0))
    def _init():
        acc_ref[...] = jnp.zeros_like(acc_ref)

    rs = jnp.sum(x1, axis=1, keepdims=True)
    rss = jnp.sum(x1 * x1, axis=1, keepdims=True)
    acc_ref[...] += jnp.concatenate([rs, rss], axis=1)


def _pass2_kernel(x1_ref, sc_ref, bi_ref, w2_ref, x2_ref, acc_ref):
    b = pl.program_id(0)
    t = pl.program_id(1)
    y = jnp.maximum(x1_ref[0].astype(jnp.float32) * sc_ref[...] + bi_ref[...],
                    0.0)
    x2 = jnp.dot(w2_ref[...], y.astype(jnp.bfloat16),
                 preferred_element_type=jnp.float32)
    x2_ref[0] = x2.astype(jnp.bfloat16)

    @pl.when(jnp.logical_and(b == 0, t == 0))
    def _init():
        acc_ref[...] = jnp.zeros_like(acc_ref)

    rs = jnp.sum(x2, axis=1, keepdims=True)
    rss = jnp.sum(x2 * x2, axis=1, keepdims=True)
    acc_ref[...] += jnp.concatenate([rs, rss], axis=1)


def _pass3_kernel(x2_ref, sc_ref, bi_ref, out_ref):
    out_ref[0] = jnp.maximum(
        x2_ref[0].astype(jnp.float32) * sc_ref[...] + bi_ref[...], 0.0)


def _bn_coeffs(acc, ntot, g, b, eps=1e-5):
    mean = acc[:, 0] / ntot
    var = acc[:, 1] / ntot - mean * mean
    sc = g / jnp.sqrt(var + eps)
    bi = b - mean * sc
    return sc[:, None], bi[:, None]


@functools.partial(jax.jit, static_argnames=())
def kernel(unknown, known, unknow_feats, known_feats, W1, g1, b1, W2, g2, b2):
    B, n, _ = unknown.shape
    m = known.shape[1]
    c2 = known_feats.shape[1]
    c1 = unknow_feats.shape[1]
    co1 = W1.shape[0]
    co2 = W2.shape[0]
    tn = _TN
    nt = n // tn
    ntot = jnp.float32(B * n)

    unk_t = jnp.transpose(unknown, (0, 2, 1))
    uf_bf = unknow_feats.astype(jnp.bfloat16)
    w1_bf = W1.astype(jnp.bfloat16)
    w2_bf = W2.astype(jnp.bfloat16)

    grid = (B, nt)
    x1, acc1 = pl.pallas_call(
        functools.partial(_pass1_kernel, m=m),
        grid=grid,
        in_specs=[
            pl.BlockSpec((1, 3, tn), lambda b, t: (b, 0, t)),
            pl.BlockSpec((1, m, 3), lambda b, t: (b, 0, 0)),
            pl.BlockSpec((1, c2, m), lambda b, t: (b, 0, 0)),
            pl.BlockSpec((1, c1, tn), lambda b, t: (b, 0, t)),
            pl.BlockSpec((co1, c1 + c2), lambda b, t: (0, 0)),
        ],
        out_specs=[
            pl.BlockSpec((1, co1, tn), lambda b, t: (b, 0, t)),
            pl.BlockSpec((co1, 2), lambda b, t: (0, 0)),
        ],
        out_shape=[
            jax.ShapeDtypeStruct((B, co1, n), jnp.bfloat16),
            jax.ShapeDtypeStruct((co1, 2), jnp.float32),
        ],
    )(unk_t, known, known_feats, uf_bf, w1_bf)

    sc1, bi1 = _bn_coeffs(acc1, ntot, g1, b1)

    x2, acc2 = pl.pallas_call(
        _pass2_kernel,
        grid=grid,
        in_specs=[
            pl.BlockSpec((1, co1, tn), lambda b, t: (b, 0, t)),
            pl.BlockSpec((co1, 1), lambda b, t: (0, 0)),
            pl.BlockSpec((co1, 1), lambda b, t: (0, 0)),
            pl.BlockSpec((co2, co1), lambda b, t: (0, 0)),
        ],
        out_specs=[
            pl.BlockSpec((1, co2, tn), lambda b, t: (b, 0, t)),
            pl.BlockSpec((co2, 2), lambda b, t: (0, 0)),
        ],
        out_shape=[
            jax.ShapeDtypeStruct((B, co2, n), jnp.bfloat16),
            jax.ShapeDtypeStruct((co2, 2), jnp.float32),
        ],
    )(x1, sc1, bi1, w2_bf)

    sc2, bi2 = _bn_coeffs(acc2, ntot, g2, b2)

    out = pl.pallas_call(
        _pass3_kernel,
        grid=grid,
        in_specs=[
            pl.BlockSpec((1, co2, tn), lambda b, t: (b, 0, t)),
            pl.BlockSpec((co2, 1), lambda b, t: (0, 0)),
            pl.BlockSpec((co2, 1), lambda b, t: (0, 0)),
        ],
        out_specs=pl.BlockSpec((1, co2, tn), lambda b, t: (b, 0, t)),
        out_shape=jax.ShapeDtypeStruct((B, co2, n), jnp.float32),
    )(x2, sc2, bi2)

    return out
